# baseline (speedup 1.0000x reference)
import jax
import jax.numpy as jnp
from jax import lax
from jax.experimental import pallas as pl
from jax.experimental.pallas import tpu as pltpu

NUM_EXPERTS = 64
TOP_K = 2
HIDDEN_DIM = 768
INTERMEDIATE_DIM = 2048
AUX_LOSS_COEF = 0.01
Z_LOSS_COEF = 0.001

BLOCK_ROWS = 128
CHUNK = 256


def _col_cumsum_excl(m):
    r, c = m.shape
    ii = lax.broadcasted_iota(jnp.int32, (CHUNK, CHUNK), 0)
    jj = lax.broadcasted_iota(jnp.int32, (CHUNK, CHUNK), 1)
    ltri = (jj < ii).astype(jnp.float32)
    pieces = []
    prefix = jnp.zeros((1, c), jnp.float32)
    for s in range(0, r, CHUNK):
        blk = m[s:s + CHUNK]
        cs = jnp.dot(ltri, blk, preferred_element_type=jnp.float32) + prefix
        pieces.append(cs)
        prefix = prefix + jnp.sum(blk, axis=0, keepdims=True)
    return jnp.concatenate(pieces, axis=0)


def _router_kernel(x_ref, gw_ref, lb_ref, z_ref,
                   pair_k_ref, pair_w_ref, sched_ref):
    x = x_ref[...]
    gw = gw_ref[...]
    logits = lax.dot_general(x, gw, (((1,), (1,)), ((), ())),
                             preferred_element_type=jnp.float32)
    m = jnp.max(logits, axis=-1, keepdims=True)
    ex = jnp.exp(logits - m)
    s = jnp.sum(ex, axis=-1, keepdims=True)
    probs = ex / s

    n, e = probs.shape
    iota = lax.broadcasted_iota(jnp.int32, (n, e), 1)
    p1 = jnp.max(probs, axis=-1, keepdims=True)
    i1 = jnp.min(jnp.where(probs == p1, iota, e), axis=-1, keepdims=True)
    mask1 = iota == i1
    probs_m = jnp.where(mask1, -1.0, probs)
    p2 = jnp.max(probs_m, axis=-1, keepdims=True)
    i2 = jnp.min(jnp.where(probs_m == p2, iota, e), axis=-1, keepdims=True)
    mask2 = iota == i2

    wsum = p1 + p2
    w1 = p1 / wsum
    w2 = p2 / wsum

    one_hot = (mask1 | mask2).astype(jnp.float32)
    tokens_per_expert = jnp.mean(one_hot, axis=0)
    prob_per_expert = jnp.mean(probs, axis=0)
    lb = (NUM_EXPERTS * jnp.sum(tokens_per_expert * prob_per_expert)
          * AUX_LOSS_COEF)
    lse = m[:, 0] + jnp.log(s[:, 0])
    z = jnp.mean(lse * lse) * Z_LOSS_COEF
    lb_ref[...] = jnp.reshape(lb, (1, 1))
    z_ref[...] = jnp.reshape(z, (1, 1))

    oh1 = mask1.astype(jnp.float32)
    oh2 = mask2.astype(jnp.float32)
    oh12 = jnp.concatenate([oh1, oh2], axis=1)
    c12 = _col_cumsum_excl(oh12)
    c1 = c12[:, :e]
    c2 = c12[:, e:]
    s1 = jnp.sum(oh1, axis=0, keepdims=True)
    counts = s1 + jnp.sum(oh2, axis=0, keepdims=True)
    rank1 = jnp.sum(c1 * oh1, axis=-1, keepdims=True)
    rank2 = jnp.sum((c2 + s1) * oh2, axis=-1, keepdims=True)

    key1 = i1 * 4096 + rank1.astype(jnp.int32)
    key2 = i2 * 4096 + rank2.astype(jnp.int32)
    pair_k_ref[0:1, :n] = jnp.reshape(key1, (1, n))
    pair_k_ref[0:1, n:] = jnp.reshape(key2, (1, n))
    pair_w_ref[0:1, :n] = jnp.reshape(w1, (1, n))
    pair_w_ref[0:1, n:] = jnp.reshape(w2, (1, n))

    g_max = sched_ref.shape[0]
    b = BLOCK_ROWS
    nblk = jnp.floor((counts + (b - 1)) * (1.0 / b))
    ecum_i = jnp.dot(nblk, (lax.broadcasted_iota(jnp.int32, (e, e), 0)
                            <= lax.broadcasted_iota(jnp.int32, (e, e), 1)
                            ).astype(jnp.float32),
                     preferred_element_type=jnp.float32)
    blk_off = ecum_i - nblk
    gv = lax.broadcasted_iota(jnp.int32, (g_max, e), 0).astype(jnp.float32)
    e_of_g = jnp.sum((ecum_i <= gv).astype(jnp.float32), axis=-1,
                     keepdims=True)
    e_of_g = jnp.minimum(e_of_g, float(e - 1))
    eiota = lax.broadcasted_iota(jnp.int32, (g_max, e), 1).astype(jnp.float32)
    pick = (eiota == e_of_g).astype(jnp.float32)
    off_g = jnp.sum(pick * blk_off, axis=-1, keepdims=True)
    cnt_g = jnp.sum(pick * counts, axis=-1, keepdims=True)
    j_g = gv[:, 0:1] - off_g
    nvalid = jnp.clip(cnt_g - j_g * b, 0.0, float(b))
    sched_ref[:, 0:1] = e_of_g.astype(jnp.int32)
    sched_ref[:, 1:2] = j_g.astype(jnp.int32)
    sched_ref[:, 2:3] = nvalid.astype(jnp.int32)


def _ffn_kernel(sched_ref, x_ref, wg_ref, wu_ref, wd_ref,
                pk_ref, pw_ref, out_ref):
    g = pl.program_id(0)

    @pl.when(g == 0)
    def _init():
        out_ref[...] = jnp.zeros_like(out_ref)

    @pl.when(sched_ref[g, 2] > 0)
    def _compute():
        e_g = sched_ref[g, 0]
        j_g = sched_ref[g, 1]
        n = x_ref.shape[0]
        b = BLOCK_ROWS
        k0 = pk_ref[0:1, :n]
        k1 = pk_ref[0:1, n:]
        w0 = pw_ref[0:1, :n]
        w1 = pw_ref[0:1, n:]
        tgt = (e_g * 4096 + j_g * b
               + lax.broadcasted_iota(jnp.int32, (b, 1), 0))
        m0 = k0 == tgt
        m1 = k1 == tgt
        oh = (m0 | m1).astype(jnp.float32)
        wcol = jnp.max(jnp.where(m0, w0, jnp.where(m1, w1, 0.0)),
                       axis=1, keepdims=True)
        xg = jnp.dot(oh, x_ref[...], preferred_element_type=jnp.float32)
        wg = wg_ref[0]
        wu = wu_ref[0]
        wd = wd_ref[0]
        a = lax.dot_general(xg, wg, (((1,), (1,)), ((), ())),
                            preferred_element_type=jnp.float32)
        u = lax.dot_general(xg, wu, (((1,), (1,)), ((), ())),
                            preferred_element_type=jnp.float32)
        h = (a * jax.nn.sigmoid(a)) * u
        y = lax.dot_general(h, wd, (((1,), (1,)), ((), ())),
                            preferred_element_type=jnp.float32)
        y = y * wcol
        out_ref[...] += lax.dot_general(oh, y, (((0,), (0,)), ((), ())),
                                        preferred_element_type=jnp.float32)


@jax.jit
def kernel(hidden_states, gate_w, w_gate, w_up, w_down):
    bsz, seq, d = hidden_states.shape
    n = bsz * seq
    e = NUM_EXPERTS
    k = TOP_K
    p = n * k
    b = BLOCK_ROWS
    g_max = p // b + e
    x = hidden_states.reshape(n, d)

    lb, z, pair_k, pair_w, sched = pl.pallas_call(
        _router_kernel,
        out_shape=(
            jax.ShapeDtypeStruct((1, 1), jnp.float32),
            jax.ShapeDtypeStruct((1, 1), jnp.float32),
            jax.ShapeDtypeStruct((1, p), jnp.int32),
            jax.ShapeDtypeStruct((1, p), jnp.float32),
            jax.ShapeDtypeStruct((g_max, 3), jnp.int32),
        ),
    )(x, gate_w)

    grid_spec = pltpu.PrefetchScalarGridSpec(
        num_scalar_prefetch=1,
        grid=(g_max,),
        in_specs=[
            pl.BlockSpec((n, d), lambda g, s: (0, 0)),
            pl.BlockSpec((1, INTERMEDIATE_DIM, d), lambda g, s: (s[g, 0], 0, 0)),
            pl.BlockSpec((1, INTERMEDIATE_DIM, d), lambda g, s: (s[g, 0], 0, 0)),
            pl.BlockSpec((1, d, INTERMEDIATE_DIM), lambda g, s: (s[g, 0], 0, 0)),
            pl.BlockSpec((1, p), lambda g, s: (0, 0)),
            pl.BlockSpec((1, p), lambda g, s: (0, 0)),
        ],
        out_specs=pl.BlockSpec((n, d), lambda g, s: (0, 0)),
    )
    out = pl.pallas_call(
        _ffn_kernel,
        grid_spec=grid_spec,
        out_shape=jax.ShapeDtypeStruct((n, d), jnp.float32),
        compiler_params=pltpu.CompilerParams(
            dimension_semantics=("arbitrary",),
        ),
    )(sched, x, w_gate, w_up, w_down, pair_k, pair_w)

    return (out.reshape(bsz, seq, d),
            lb.reshape(()), z.reshape(()))


# speedup vs baseline: 1.0125x; 1.0125x over previous
import jax
import jax.numpy as jnp
from jax import lax
from jax.experimental import pallas as pl
from jax.experimental.pallas import tpu as pltpu

NUM_EXPERTS = 64
TOP_K = 2
HIDDEN_DIM = 768
INTERMEDIATE_DIM = 2048
AUX_LOSS_COEF = 0.01
Z_LOSS_COEF = 0.001

BLOCK_ROWS = 128
CHUNK = 256


def _row_cumsum_excl(m):
    r, c = m.shape
    ii = lax.broadcasted_iota(jnp.int32, (CHUNK, CHUNK), 0)
    jj = lax.broadcasted_iota(jnp.int32, (CHUNK, CHUNK), 1)
    utri = (ii < jj).astype(jnp.float32)
    pieces = []
    prefix = jnp.zeros((r, 1), jnp.float32)
    for s in range(0, c, CHUNK):
        blk = m[:, s:s + CHUNK]
        cs = jnp.dot(blk, utri, preferred_element_type=jnp.float32) + prefix
        pieces.append(cs)
        prefix = prefix + jnp.sum(blk, axis=1, keepdims=True)
    return jnp.concatenate(pieces, axis=1)


def _router_kernel(x_ref, gw_ref, lb_ref, z_ref,
                   pair_k_ref, pair_w_ref, sched_ref):
    x = x_ref[...]
    gw = gw_ref[...]
    logits = lax.dot_general(gw, x, (((1,), (1,)), ((), ())),
                             preferred_element_type=jnp.float32)
    m = jnp.max(logits, axis=0, keepdims=True)
    ex = jnp.exp(logits - m)
    s = jnp.sum(ex, axis=0, keepdims=True)
    probs = ex / s

    e, n = probs.shape
    iota = lax.broadcasted_iota(jnp.int32, (e, n), 0)
    p1 = jnp.max(probs, axis=0, keepdims=True)
    i1 = jnp.min(jnp.where(probs == p1, iota, e), axis=0, keepdims=True)
    mask1 = iota == i1
    probs_m = jnp.where(mask1, -1.0, probs)
    p2 = jnp.max(probs_m, axis=0, keepdims=True)
    i2 = jnp.min(jnp.where(probs_m == p2, iota, e), axis=0, keepdims=True)
    mask2 = iota == i2

    wsum = p1 + p2
    w1 = p1 / wsum
    w2 = p2 / wsum

    one_hot = (mask1 | mask2).astype(jnp.float32)
    tokens_per_expert = jnp.mean(one_hot, axis=1, keepdims=True)
    prob_per_expert = jnp.mean(probs, axis=1, keepdims=True)
    lb = (NUM_EXPERTS * jnp.sum(tokens_per_expert * prob_per_expert)
          * AUX_LOSS_COEF)
    lse = m + jnp.log(s)
    z = jnp.mean(lse * lse) * Z_LOSS_COEF
    lb_ref[...] = jnp.reshape(lb, (1, 1))
    z_ref[...] = jnp.reshape(z, (1, 1))

    oh1 = mask1.astype(jnp.float32)
    oh2 = mask2.astype(jnp.float32)
    oh12 = jnp.concatenate([oh1, oh2], axis=0)
    c12 = _row_cumsum_excl(oh12)
    c1 = c12[:e]
    c2 = c12[e:]
    s1 = jnp.sum(oh1, axis=1, keepdims=True)
    counts = s1 + jnp.sum(oh2, axis=1, keepdims=True)
    rank1 = jnp.sum(c1 * oh1, axis=0, keepdims=True)
    rank2 = jnp.sum((c2 + s1) * oh2, axis=0, keepdims=True)

    pair_k_ref[0:1, :n] = i1 * 4096 + rank1.astype(jnp.int32)
    pair_k_ref[0:1, n:] = i2 * 4096 + rank2.astype(jnp.int32)
    pair_w_ref[0:1, :n] = w1
    pair_w_ref[0:1, n:] = w2

    g_max = sched_ref.shape[1]
    b = BLOCK_ROWS
    nblk = jnp.floor((counts + (b - 1)) * (1.0 / b))
    lincl = (lax.broadcasted_iota(jnp.int32, (e, e), 1)
             <= lax.broadcasted_iota(jnp.int32, (e, e), 0)
             ).astype(jnp.float32)
    ecum_i = jnp.dot(lincl, nblk,
                     preferred_element_type=jnp.float32)
    blk_off = ecum_i - nblk
    gv = lax.broadcasted_iota(jnp.int32, (1, g_max), 1).astype(jnp.float32)
    e_of_g = jnp.sum((ecum_i <= gv).astype(jnp.float32), axis=0,
                     keepdims=True)
    e_of_g = jnp.minimum(e_of_g, float(e - 1))
    eiota = lax.broadcasted_iota(jnp.int32, (e, g_max), 0).astype(jnp.float32)
    pick = (eiota == e_of_g).astype(jnp.float32)
    off_g = jnp.sum(pick * blk_off, axis=0, keepdims=True)
    cnt_g = jnp.sum(pick * counts, axis=0, keepdims=True)
    j_g = gv - off_g
    nvalid = jnp.clip(cnt_g - j_g * b, 0.0, float(b))
    sched_ref[0:1, :] = e_of_g.astype(jnp.int32)
    sched_ref[1:2, :] = j_g.astype(jnp.int32)
    sched_ref[2:3, :] = nvalid.astype(jnp.int32)


def _ffn_kernel(sched_ref, x_ref, wg_ref, wu_ref, wd_ref,
                pk_ref, pw_ref, out_ref):
    g = pl.program_id(0)

    @pl.when(g == 0)
    def _init():
        out_ref[...] = jnp.zeros_like(out_ref)

    @pl.when(sched_ref[2, g] > 0)
    def _compute():
        e_g = sched_ref[0, g]
        j_g = sched_ref[1, g]
        n = x_ref.shape[0]
        b = BLOCK_ROWS
        k0 = pk_ref[0:1, :n]
        k1 = pk_ref[0:1, n:]
        w0 = pw_ref[0:1, :n]
        w1 = pw_ref[0:1, n:]
        tgt = (e_g * 4096 + j_g * b
               + lax.broadcasted_iota(jnp.int32, (b, 1), 0))
        m0 = k0 == tgt
        m1 = k1 == tgt
        oh = (m0 | m1).astype(jnp.float32)
        wcol = jnp.max(jnp.where(m0, w0, jnp.where(m1, w1, 0.0)),
                       axis=1, keepdims=True)
        xg = jnp.dot(oh, x_ref[...], preferred_element_type=jnp.float32)
        wg = wg_ref[0]
        wu = wu_ref[0]
        wd = wd_ref[0]
        a = lax.dot_general(xg, wg, (((1,), (1,)), ((), ())),
                            preferred_element_type=jnp.float32)
        u = lax.dot_general(xg, wu, (((1,), (1,)), ((), ())),
                            preferred_element_type=jnp.float32)
        h = (a * jax.nn.sigmoid(a)) * u
        y = lax.dot_general(h, wd, (((1,), (1,)), ((), ())),
                            preferred_element_type=jnp.float32)
        y = y * wcol
        out_ref[...] += lax.dot_general(oh, y, (((0,), (0,)), ((), ())),
                                        preferred_element_type=jnp.float32)


@jax.jit
def kernel(hidden_states, gate_w, w_gate, w_up, w_down):
    bsz, seq, d = hidden_states.shape
    n = bsz * seq
    e = NUM_EXPERTS
    k = TOP_K
    p = n * k
    b = BLOCK_ROWS
    g_max = p // b + e
    x = hidden_states.reshape(n, d)

    lb, z, pair_k, pair_w, sched = pl.pallas_call(
        _router_kernel,
        out_shape=(
            jax.ShapeDtypeStruct((1, 1), jnp.float32),
            jax.ShapeDtypeStruct((1, 1), jnp.float32),
            jax.ShapeDtypeStruct((1, p), jnp.int32),
            jax.ShapeDtypeStruct((1, p), jnp.float32),
            jax.ShapeDtypeStruct((3, g_max), jnp.int32),
        ),
    )(x, gate_w)

    grid_spec = pltpu.PrefetchScalarGridSpec(
        num_scalar_prefetch=1,
        grid=(g_max,),
        in_specs=[
            pl.BlockSpec((n, d), lambda g, s: (0, 0)),
            pl.BlockSpec((1, INTERMEDIATE_DIM, d), lambda g, s: (s[0, g], 0, 0)),
            pl.BlockSpec((1, INTERMEDIATE_DIM, d), lambda g, s: (s[0, g], 0, 0)),
            pl.BlockSpec((1, d, INTERMEDIATE_DIM), lambda g, s: (s[0, g], 0, 0)),
            pl.BlockSpec((1, p), lambda g, s: (0, 0)),
            pl.BlockSpec((1, p), lambda g, s: (0, 0)),
        ],
        out_specs=pl.BlockSpec((n, d), lambda g, s: (0, 0)),
    )
    out = pl.pallas_call(
        _ffn_kernel,
        grid_spec=grid_spec,
        out_shape=jax.ShapeDtypeStruct((n, d), jnp.float32),
        compiler_params=pltpu.CompilerParams(
            dimension_semantics=("arbitrary",),
        ),
    )(sched, x, w_gate, w_up, w_down, pair_k, pair_w)

    return (out.reshape(bsz, seq, d),
            lb.reshape(()), z.reshape(()))
